# baseline (speedup 1.0000x reference)
import functools

import jax
import jax.numpy as jnp
from jax import lax
from jax.experimental import pallas as pl
from jax.experimental.pallas import tpu as pltpu
from jax.experimental.pallas import tpu_sc as plsc

TV = 2048


@functools.cache
def _sc_gather(num_rows: int, vocab: int, embed: int):
    info = plsc.get_sparse_core_info()
    nw = info.num_cores * info.num_subcores
    bpw = num_rows // nw
    chunk = 128
    nch = (bpw + chunk - 1) // chunk
    mesh = plsc.VectorSubcoreMesh(core_axis_name="c", subcore_axis_name="s")

    @functools.partial(
        pl.kernel,
        mesh=mesh,
        compiler_params=pltpu.CompilerParams(use_tc_tiling_on_sc=False),
        out_type=jax.ShapeDtypeStruct((num_rows, embed), jnp.float32),
        scratch_types=[
            pltpu.VMEM((bpw,), jnp.int32),
            pltpu.VMEM((bpw, embed), jnp.float32),
            pltpu.SemaphoreType.DMA,
        ],
    )
    def gather_kernel(idx_hbm, table_hbm, out_hbm, idx_v, rows_v, sem):
        wid = lax.axis_index("s") * info.num_cores + lax.axis_index("c")
        base = wid * bpw
        pltpu.sync_copy(idx_hbm.at[pl.ds(base, bpw)], idx_v)
        copies = []
        for c in range(nch):
            off = c * chunk
            sz = min(chunk, bpw - off)
            copies.append(
                pltpu.async_copy(
                    table_hbm.at[idx_v.at[pl.ds(off, sz)]],
                    rows_v.at[pl.ds(off, sz)],
                    sem,
                )
            )
        for cp in copies:
            cp.wait()
        pltpu.sync_copy(rows_v, out_hbm.at[pl.ds(base, bpw)])

    return gather_kernel


def _fc1(gathered, W1, b1, batch, ctx2, embed):
    hidden = W1.shape[1]

    def body(x_ref, w_ref, b_ref, h_ref, acc_ref):
        c = pl.program_id(0)

        @pl.when(c == 0)
        def _():
            acc_ref[...] = jnp.zeros((batch, hidden), jnp.float32)

        acc_ref[...] += jnp.dot(
            x_ref[...].astype(jnp.bfloat16),
            w_ref[...].astype(jnp.bfloat16),
            preferred_element_type=jnp.float32,
        )
        h_ref[...] = jnp.maximum(acc_ref[...] + b_ref[...], 0.0).astype(
            jnp.bfloat16
        )

    return pl.pallas_call(
        body,
        grid=(ctx2,),
        in_specs=[
            pl.BlockSpec((batch, embed), lambda c: (c, 0)),
            pl.BlockSpec((embed, hidden), lambda c: (c, 0)),
            pl.BlockSpec((1, hidden), lambda c: (0, 0)),
        ],
        out_specs=pl.BlockSpec((batch, hidden), lambda c: (0, 0)),
        out_shape=jax.ShapeDtypeStruct((batch, hidden), jnp.bfloat16),
        scratch_shapes=[pltpu.VMEM((batch, hidden), jnp.float32)],
    )(gathered, W1, b1.reshape(1, hidden))


def _logits_pass(h_bf, W2, b2):
    batch, hidden = h_bf.shape
    vocab = W2.shape[1]
    nj = (vocab + TV - 1) // TV
    padded = ((vocab + 4095) // 4096) * 4096

    def body(h_ref, w_ref, b_ref, l_ref, lse_ref, m_ref, s_ref):
        j = pl.program_id(0)

        @pl.when(j == 0)
        def _():
            m_ref[...] = jnp.full((batch, 128), -jnp.inf, jnp.float32)
            s_ref[...] = jnp.zeros((batch, 128), jnp.float32)

        logits = (
            jnp.dot(
                h_ref[...],
                w_ref[...].astype(jnp.bfloat16),
                preferred_element_type=jnp.float32,
            )
            + b_ref[...]
        )
        l_ref[...] = logits.astype(jnp.bfloat16)
        col = j * TV + lax.broadcasted_iota(jnp.int32, (1, TV), 1)
        logits = jnp.where(col < vocab, logits, -jnp.inf)

        chunks = [logits[:, k * 128 : (k + 1) * 128] for k in range(TV // 128)]
        bm = chunks[0]
        for c in chunks[1:]:
            bm = jnp.maximum(bm, c)
        m_old = m_ref[...]
        m_new = jnp.maximum(m_old, bm)
        s = s_ref[...] * jnp.exp(m_old - m_new)
        for c in chunks:
            s = s + jnp.exp(c - m_new)
        m_ref[...] = m_new
        s_ref[...] = s

        @pl.when(j == nj - 1)
        def _():
            big = jnp.max(m_new, axis=1, keepdims=True)
            tot = jnp.sum(s * jnp.exp(m_new - big), axis=1, keepdims=True)
            lse_ref[...] = big + jnp.log(tot)

    return pl.pallas_call(
        body,
        grid=(nj,),
        in_specs=[
            pl.BlockSpec((batch, hidden), lambda j: (0, 0)),
            pl.BlockSpec((hidden, TV), lambda j: (0, j)),
            pl.BlockSpec((1, TV), lambda j: (0, j)),
        ],
        out_specs=[
            pl.BlockSpec((batch, TV), lambda j: (0, j)),
            pl.BlockSpec((batch, 1), lambda j: (0, 0)),
        ],
        out_shape=[
            jax.ShapeDtypeStruct((batch, padded), jnp.bfloat16),
            jax.ShapeDtypeStruct((batch, 1), jnp.float32),
        ],
        scratch_shapes=[
            pltpu.VMEM((batch, 128), jnp.float32),
            pltpu.VMEM((batch, 128), jnp.float32),
        ],
    )(h_bf, W2, b2.reshape(1, vocab))


def _fixup_pass(logits_bf, lse, vocab):
    batch, padded = logits_bf.shape
    pw = ((vocab + 127) // 128) * 128
    rb = 16
    nch = batch // rb
    nbuf = 4

    def body(l_hbm, lse_ref, o_hbm, vin, vout, sem_in, sem_out):
        def in_copy(i, b):
            return pltpu.make_async_copy(
                l_hbm.at[pl.ds(i * rb, rb), pl.ds(0, pw)],
                vin.at[b],
                sem_in.at[b],
            )

        def out_copy(i, b):
            return pltpu.make_async_copy(
                vout.at[b],
                o_hbm.at[pl.ds(i * rb, rb), :],
                sem_out.at[b],
            )

        for i in range(nbuf):
            in_copy(i, i).start()
        for i in range(nch):
            b = i % nbuf
            if i >= nbuf:
                out_copy(i - nbuf, b).wait()
            in_copy(i, b).wait()
            lse_c = lse_ref[pl.ds(i * rb, rb), :]
            vout[b] = (
                vin[b][:, :vocab].astype(jnp.float32) - lse_c
            )
            out_copy(i, b).start()
            nxt = i + nbuf
            if nxt < nch:
                in_copy(nxt, b).start()
        for i in range(nch - nbuf, nch):
            out_copy(i, i % nbuf).wait()

    return pl.pallas_call(
        body,
        in_specs=[
            pl.BlockSpec(memory_space=pl.ANY),
            pl.BlockSpec(memory_space=pltpu.VMEM),
        ],
        out_specs=pl.BlockSpec(memory_space=pl.ANY),
        out_shape=jax.ShapeDtypeStruct((batch, vocab), jnp.float32),
        scratch_shapes=[
            pltpu.VMEM((nbuf, rb, pw), jnp.bfloat16),
            pltpu.VMEM((nbuf, rb, vocab), jnp.float32),
            pltpu.SemaphoreType.DMA((nbuf,)),
            pltpu.SemaphoreType.DMA((nbuf,)),
        ],
    )(logits_bf, lse)


def kernel(inputs, emb, W1, b1, W2, b2):
    batch, ctx2 = inputs.shape
    vocab, embed = emb.shape
    idx = inputs.astype(jnp.int32).T.reshape(-1)
    gathered = _sc_gather(batch * ctx2, vocab, embed)(idx, emb)
    h = _fc1(gathered, W1, b1, batch, ctx2, embed)
    logits_bf, lse = _logits_pass(h, W2, b2)
    return _fixup_pass(logits_bf, lse, vocab)


# speedup vs baseline: 1.0065x; 1.0065x over previous
import functools

import jax
import jax.numpy as jnp
from jax import lax
from jax.experimental import pallas as pl
from jax.experimental.pallas import tpu as pltpu
from jax.experimental.pallas import tpu_sc as plsc

TV = 2048


@functools.cache
def _sc_gather(num_rows: int, vocab: int, embed: int):
    info = plsc.get_sparse_core_info()
    nw = info.num_cores * info.num_subcores
    bpw = num_rows // nw
    chunk = 128
    nch = (bpw + chunk - 1) // chunk
    mesh = plsc.VectorSubcoreMesh(core_axis_name="c", subcore_axis_name="s")

    @functools.partial(
        pl.kernel,
        mesh=mesh,
        compiler_params=pltpu.CompilerParams(use_tc_tiling_on_sc=False),
        out_type=jax.ShapeDtypeStruct((num_rows, embed), jnp.float32),
        scratch_types=[
            pltpu.VMEM((bpw,), jnp.int32),
            pltpu.VMEM((bpw, embed), jnp.float32),
            pltpu.SemaphoreType.DMA,
        ],
    )
    def gather_kernel(idx_hbm, table_hbm, out_hbm, idx_v, rows_v, sem):
        wid = lax.axis_index("s") * info.num_cores + lax.axis_index("c")
        base = wid * bpw
        pltpu.sync_copy(idx_hbm.at[pl.ds(base, bpw)], idx_v)
        copies = []
        for c in range(nch):
            off = c * chunk
            sz = min(chunk, bpw - off)
            copies.append(
                pltpu.async_copy(
                    table_hbm.at[idx_v.at[pl.ds(off, sz)]],
                    rows_v.at[pl.ds(off, sz)],
                    sem,
                )
            )
        for cp in copies:
            cp.wait()
        pltpu.sync_copy(rows_v, out_hbm.at[pl.ds(base, bpw)])

    return gather_kernel


def _mlp_softmax(gathered, W1, b1, W2, b2, batch, ctx2, embed):
    hidden = W1.shape[1]
    vocab = W2.shape[1]
    nj = (vocab + TV - 1) // TV
    p2, p3 = ctx2, ctx2 + nj
    steps = ctx2 + 2 * nj

    def vocab_j(c):
        return jnp.where(c < p3, jnp.maximum(c - p2, 0), c - p3)

    def body(x_ref, w1_ref, b1_ref, w2_ref, b2_ref, o_ref,
             acc_ref, h_ref, m_ref, s_ref, lse_ref):
        c = pl.program_id(0)

        @pl.when(c == 0)
        def _():
            acc_ref[...] = jnp.zeros((batch, hidden), jnp.float32)

        @pl.when(c < p2)
        def _():
            acc_ref[...] += jnp.dot(
                x_ref[...].astype(jnp.bfloat16),
                w1_ref[...].astype(jnp.bfloat16),
                preferred_element_type=jnp.float32,
            )

        @pl.when(c == p2 - 1)
        def _():
            h_ref[...] = jnp.maximum(
                acc_ref[...] + b1_ref[...], 0.0
            ).astype(jnp.bfloat16)
            m_ref[...] = jnp.full((batch, 128), -jnp.inf, jnp.float32)
            s_ref[...] = jnp.zeros((batch, 128), jnp.float32)

        @pl.when((c >= p2) & (c < p3))
        def _():
            j = c - p2
            logits = (
                jnp.dot(
                    h_ref[...],
                    w2_ref[...].astype(jnp.bfloat16),
                    preferred_element_type=jnp.float32,
                )
                + b2_ref[...]
            )
            col = j * TV + lax.broadcasted_iota(jnp.int32, (1, TV), 1)
            logits = jnp.where(col < vocab, logits, -jnp.inf)
            chunks = [
                logits[:, k * 128 : (k + 1) * 128] for k in range(TV // 128)
            ]
            bm = chunks[0]
            for ch in chunks[1:]:
                bm = jnp.maximum(bm, ch)
            m_old = m_ref[...]
            m_new = jnp.maximum(m_old, bm)
            sval = s_ref[...] * jnp.exp(m_old - m_new)
            for ch in chunks:
                sval = sval + jnp.exp(ch - m_new)
            m_ref[...] = m_new
            s_ref[...] = sval

            @pl.when(c == p3 - 1)
            def _():
                big = jnp.max(m_new, axis=1, keepdims=True)
                tot = jnp.sum(
                    sval * jnp.exp(m_new - big), axis=1, keepdims=True
                )
                lse_ref[...] = big + jnp.log(tot)

        @pl.when(c >= p3)
        def _():
            logits = (
                jnp.dot(
                    h_ref[...],
                    w2_ref[...].astype(jnp.bfloat16),
                    preferred_element_type=jnp.float32,
                )
                + b2_ref[...]
            )
            o_ref[...] = logits - lse_ref[...]

    return pl.pallas_call(
        body,
        grid=(steps,),
        in_specs=[
            pl.BlockSpec((batch, embed), lambda c: (jnp.minimum(c, p2 - 1), 0)),
            pl.BlockSpec((embed, hidden), lambda c: (jnp.minimum(c, p2 - 1), 0)),
            pl.BlockSpec((1, hidden), lambda c: (0, 0)),
            pl.BlockSpec((hidden, TV), lambda c: (0, vocab_j(c))),
            pl.BlockSpec((1, TV), lambda c: (0, vocab_j(c))),
        ],
        out_specs=pl.BlockSpec(
            (batch, TV), lambda c: (0, jnp.where(c < p3, 0, c - p3))
        ),
        out_shape=jax.ShapeDtypeStruct((batch, vocab), jnp.float32),
        scratch_shapes=[
            pltpu.VMEM((batch, hidden), jnp.float32),
            pltpu.VMEM((batch, hidden), jnp.bfloat16),
            pltpu.VMEM((batch, 128), jnp.float32),
            pltpu.VMEM((batch, 128), jnp.float32),
            pltpu.VMEM((batch, 1), jnp.float32),
        ],
    )(gathered, W1, b1.reshape(1, hidden), W2, b2.reshape(1, vocab))


def kernel(inputs, emb, W1, b1, W2, b2):
    batch, ctx2 = inputs.shape
    vocab, embed = emb.shape
    idx = inputs.astype(jnp.int32).T.reshape(-1)
    gathered = _sc_gather(batch * ctx2, vocab, embed)(idx, emb)
    return _mlp_softmax(gathered, W1, b1, W2, b2, batch, ctx2, embed)


# speedup vs baseline: 1.0194x; 1.0128x over previous
import functools

import jax
import jax.numpy as jnp
from jax import lax
from jax.experimental import pallas as pl
from jax.experimental.pallas import tpu as pltpu
from jax.experimental.pallas import tpu_sc as plsc

TV = 2048


@functools.cache
def _sc_gather(num_rows: int, vocab: int, embed: int):
    info = plsc.get_sparse_core_info()
    nw = info.num_cores * info.num_subcores
    bpw = num_rows // nw
    chunk = 128
    nch = (bpw + chunk - 1) // chunk
    mesh = plsc.VectorSubcoreMesh(core_axis_name="c", subcore_axis_name="s")

    @functools.partial(
        pl.kernel,
        mesh=mesh,
        compiler_params=pltpu.CompilerParams(use_tc_tiling_on_sc=False),
        out_type=jax.ShapeDtypeStruct((num_rows, embed), jnp.float32),
        scratch_types=[
            pltpu.VMEM((bpw,), jnp.int32),
            pltpu.VMEM((bpw, embed), jnp.float32),
            pltpu.SemaphoreType.DMA,
        ],
    )
    def gather_kernel(idx_hbm, table_hbm, out_hbm, idx_v, rows_v, sem):
        wid = lax.axis_index("s") * info.num_cores + lax.axis_index("c")
        base = wid * bpw
        pltpu.sync_copy(idx_hbm.at[pl.ds(base, bpw)], idx_v)
        copies = []
        for c in range(nch):
            off = c * chunk
            sz = min(chunk, bpw - off)
            copies.append(
                pltpu.async_copy(
                    table_hbm.at[idx_v.at[pl.ds(off, sz)]],
                    rows_v.at[pl.ds(off, sz)],
                    sem,
                )
            )
        for cp in copies:
            cp.wait()
        pltpu.sync_copy(rows_v, out_hbm.at[pl.ds(base, bpw)])

    return gather_kernel


def _mlp_softmax(gathered, W1, b1, W2, b2, batch, ctx2, embed):
    hidden = W1.shape[1]
    vocab = W2.shape[1]
    nj = (vocab + TV - 1) // TV
    nc2 = ctx2 // 2
    p2, p3 = nc2, nc2 + nj
    steps = nc2 + 2 * nj

    def vocab_j(c):
        return jnp.where(c < p3, jnp.maximum(c - p2, 0), c - p3)

    def body(x_ref, x2_ref, w1_ref, w12_ref, b1_ref, w2_ref, b2_ref, o_ref,
             acc_ref, h_ref, m_ref, s_ref, lse_ref):
        c = pl.program_id(0)

        @pl.when(c == 0)
        def _():
            acc_ref[...] = jnp.zeros((batch, hidden), jnp.float32)

        @pl.when(c < p2)
        def _():
            acc_ref[...] += jnp.dot(
                x_ref[...].astype(jnp.bfloat16),
                w1_ref[...].astype(jnp.bfloat16),
                preferred_element_type=jnp.float32,
            ) + jnp.dot(
                x2_ref[...].astype(jnp.bfloat16),
                w12_ref[...].astype(jnp.bfloat16),
                preferred_element_type=jnp.float32,
            )

        @pl.when(c == p2 - 1)
        def _():
            h_ref[...] = jnp.maximum(
                acc_ref[...] + b1_ref[...], 0.0
            ).astype(jnp.bfloat16)
            m_ref[...] = jnp.full((batch, 128), -jnp.inf, jnp.float32)
            s_ref[...] = jnp.zeros((batch, 128), jnp.float32)

        @pl.when((c >= p2) & (c < p3))
        def _():
            j = c - p2
            logits = (
                jnp.dot(
                    h_ref[...],
                    w2_ref[...].astype(jnp.bfloat16),
                    preferred_element_type=jnp.float32,
                )
                + b2_ref[...]
            )
            col = j * TV + lax.broadcasted_iota(jnp.int32, (1, TV), 1)
            logits = jnp.where(col < vocab, logits, -jnp.inf)
            chunks = [
                logits[:, k * 128 : (k + 1) * 128] for k in range(TV // 128)
            ]
            bm = chunks[0]
            for ch in chunks[1:]:
                bm = jnp.maximum(bm, ch)
            m_old = m_ref[...]
            m_new = jnp.maximum(m_old, bm)
            sval = s_ref[...] * jnp.exp(m_old - m_new)
            for ch in chunks:
                sval = sval + jnp.exp(ch - m_new)
            m_ref[...] = m_new
            s_ref[...] = sval

            @pl.when(c == p3 - 1)
            def _():
                big = jnp.max(m_new, axis=1, keepdims=True)
                tot = jnp.sum(
                    sval * jnp.exp(m_new - big), axis=1, keepdims=True
                )
                lse_ref[...] = big + jnp.log(tot)

        @pl.when(c >= p3)
        def _():
            logits = (
                jnp.dot(
                    h_ref[...],
                    w2_ref[...].astype(jnp.bfloat16),
                    preferred_element_type=jnp.float32,
                )
                + b2_ref[...]
            )
            o_ref[...] = logits - lse_ref[...]

    return pl.pallas_call(
        body,
        grid=(steps,),
        in_specs=[
            pl.BlockSpec((batch, embed), lambda c: (jnp.minimum(c, p2 - 1), 0)),
            pl.BlockSpec(
                (batch, embed), lambda c: (nc2 + jnp.minimum(c, p2 - 1), 0)
            ),
            pl.BlockSpec((embed, hidden), lambda c: (jnp.minimum(c, p2 - 1), 0)),
            pl.BlockSpec(
                (embed, hidden), lambda c: (nc2 + jnp.minimum(c, p2 - 1), 0)
            ),
            pl.BlockSpec((1, hidden), lambda c: (0, 0)),
            pl.BlockSpec((hidden, TV), lambda c: (0, vocab_j(c))),
            pl.BlockSpec((1, TV), lambda c: (0, vocab_j(c))),
        ],
        out_specs=pl.BlockSpec(
            (batch, TV), lambda c: (0, jnp.where(c < p3, 0, c - p3))
        ),
        out_shape=jax.ShapeDtypeStruct((batch, vocab), jnp.float32),
        scratch_shapes=[
            pltpu.VMEM((batch, hidden), jnp.float32),
            pltpu.VMEM((batch, hidden), jnp.bfloat16),
            pltpu.VMEM((batch, 128), jnp.float32),
            pltpu.VMEM((batch, 128), jnp.float32),
            pltpu.VMEM((batch, 1), jnp.float32),
        ],
    )(gathered, gathered, W1, W1, b1.reshape(1, hidden), W2,
      b2.reshape(1, vocab))


def kernel(inputs, emb, W1, b1, W2, b2):
    batch, ctx2 = inputs.shape
    vocab, embed = emb.shape
    idx = inputs.astype(jnp.int32).T.reshape(-1)
    gathered = _sc_gather(batch * ctx2, vocab, embed)(idx, emb)
    return _mlp_softmax(gathered, W1, b1, W2, b2, batch, ctx2, embed)


# speedup vs baseline: 1.0282x; 1.0086x over previous
import functools

import jax
import jax.numpy as jnp
from jax import lax
from jax.experimental import pallas as pl
from jax.experimental.pallas import tpu as pltpu
from jax.experimental.pallas import tpu_sc as plsc

TV = 3072


@functools.cache
def _sc_gather(num_rows: int, vocab: int, embed: int):
    info = plsc.get_sparse_core_info()
    nw = info.num_cores * info.num_subcores
    bpw = num_rows // nw
    chunk = 128
    nch = (bpw + chunk - 1) // chunk
    mesh = plsc.VectorSubcoreMesh(core_axis_name="c", subcore_axis_name="s")

    @functools.partial(
        pl.kernel,
        mesh=mesh,
        compiler_params=pltpu.CompilerParams(use_tc_tiling_on_sc=False),
        out_type=jax.ShapeDtypeStruct((num_rows, embed), jnp.float32),
        scratch_types=[
            pltpu.VMEM((bpw,), jnp.int32),
            pltpu.VMEM((bpw, embed), jnp.float32),
            pltpu.SemaphoreType.DMA,
        ],
    )
    def gather_kernel(idx_hbm, table_hbm, out_hbm, idx_v, rows_v, sem):
        wid = lax.axis_index("s") * info.num_cores + lax.axis_index("c")
        base = wid * bpw
        pltpu.sync_copy(idx_hbm.at[pl.ds(base, bpw)], idx_v)
        copies = []
        for c in range(nch):
            off = c * chunk
            sz = min(chunk, bpw - off)
            copies.append(
                pltpu.async_copy(
                    table_hbm.at[idx_v.at[pl.ds(off, sz)]],
                    rows_v.at[pl.ds(off, sz)],
                    sem,
                )
            )
        for cp in copies:
            cp.wait()
        pltpu.sync_copy(rows_v, out_hbm.at[pl.ds(base, bpw)])

    return gather_kernel


def _mlp_softmax(gathered, W1, b1, W2, b2, batch, ctx2, embed):
    hidden = W1.shape[1]
    vocab = W2.shape[1]
    nj = (vocab + TV - 1) // TV
    nc2 = ctx2 // 2
    p2, p3 = nc2, nc2 + nj
    steps = nc2 + 2 * nj

    def vocab_j(c):
        return jnp.where(c < p3, jnp.maximum(c - p2, 0), c - p3)

    def body(x_ref, x2_ref, w1_ref, w12_ref, b1_ref, w2_ref, b2_ref, o_ref,
             acc_ref, h_ref, m_ref, s_ref, lse_ref):
        c = pl.program_id(0)

        @pl.when(c == 0)
        def _():
            acc_ref[...] = jnp.zeros((batch, hidden), jnp.float32)

        @pl.when(c < p2)
        def _():
            acc_ref[...] += jnp.dot(
                x_ref[...].astype(jnp.bfloat16),
                w1_ref[...].astype(jnp.bfloat16),
                preferred_element_type=jnp.float32,
            ) + jnp.dot(
                x2_ref[...].astype(jnp.bfloat16),
                w12_ref[...].astype(jnp.bfloat16),
                preferred_element_type=jnp.float32,
            )

        @pl.when(c == p2 - 1)
        def _():
            h_ref[...] = jnp.maximum(
                acc_ref[...] + b1_ref[...], 0.0
            ).astype(jnp.bfloat16)
            m_ref[...] = jnp.full((batch, 128), -jnp.inf, jnp.float32)
            s_ref[...] = jnp.zeros((batch, 128), jnp.float32)

        @pl.when((c >= p2) & (c < p3))
        def _():
            j = c - p2
            logits = (
                jnp.dot(
                    h_ref[...],
                    w2_ref[...].astype(jnp.bfloat16),
                    preferred_element_type=jnp.float32,
                )
                + b2_ref[...]
            )
            col = j * TV + lax.broadcasted_iota(jnp.int32, (1, TV), 1)
            logits = jnp.where(col < vocab, logits, -jnp.inf)
            chunks = [
                logits[:, k * 128 : (k + 1) * 128] for k in range(TV // 128)
            ]
            bm = chunks[0]
            for ch in chunks[1:]:
                bm = jnp.maximum(bm, ch)
            m_old = m_ref[...]
            m_new = jnp.maximum(m_old, bm)
            sval = s_ref[...] * jnp.exp(m_old - m_new)
            for ch in chunks:
                sval = sval + jnp.exp(ch - m_new)
            m_ref[...] = m_new
            s_ref[...] = sval

            @pl.when(c == p3 - 1)
            def _():
                big = jnp.max(m_new, axis=1, keepdims=True)
                tot = jnp.sum(
                    sval * jnp.exp(m_new - big), axis=1, keepdims=True
                )
                lse_ref[...] = big + jnp.log(tot)

        @pl.when(c >= p3)
        def _():
            logits = (
                jnp.dot(
                    h_ref[...],
                    w2_ref[...].astype(jnp.bfloat16),
                    preferred_element_type=jnp.float32,
                )
                + b2_ref[...]
            )
            o_ref[...] = logits - lse_ref[...]

    return pl.pallas_call(
        body,
        grid=(steps,),
        in_specs=[
            pl.BlockSpec((batch, embed), lambda c: (jnp.minimum(c, p2 - 1), 0)),
            pl.BlockSpec(
                (batch, embed), lambda c: (nc2 + jnp.minimum(c, p2 - 1), 0)
            ),
            pl.BlockSpec((embed, hidden), lambda c: (jnp.minimum(c, p2 - 1), 0)),
            pl.BlockSpec(
                (embed, hidden), lambda c: (nc2 + jnp.minimum(c, p2 - 1), 0)
            ),
            pl.BlockSpec((1, hidden), lambda c: (0, 0)),
            pl.BlockSpec((hidden, TV), lambda c: (0, vocab_j(c))),
            pl.BlockSpec((1, TV), lambda c: (0, vocab_j(c))),
        ],
        out_specs=pl.BlockSpec(
            (batch, TV), lambda c: (0, jnp.where(c < p3, 0, c - p3))
        ),
        out_shape=jax.ShapeDtypeStruct((batch, vocab), jnp.float32),
        scratch_shapes=[
            pltpu.VMEM((batch, hidden), jnp.float32),
            pltpu.VMEM((batch, hidden), jnp.bfloat16),
            pltpu.VMEM((batch, 128), jnp.float32),
            pltpu.VMEM((batch, 128), jnp.float32),
            pltpu.VMEM((batch, 1), jnp.float32),
        ],
    )(gathered, gathered, W1, W1, b1.reshape(1, hidden), W2,
      b2.reshape(1, vocab))


def kernel(inputs, emb, W1, b1, W2, b2):
    batch, ctx2 = inputs.shape
    vocab, embed = emb.shape
    idx = inputs.astype(jnp.int32).T.reshape(-1)
    gathered = _sc_gather(batch * ctx2, vocab, embed)(idx, emb)
    return _mlp_softmax(gathered, W1, b1, W2, b2, batch, ctx2, embed)
